# baseline (speedup 1.0000x reference)
import functools

import jax
import jax.numpy as jnp
from jax import lax
from jax.experimental import pallas as pl
from jax.experimental.pallas import tpu as pltpu
from jax.experimental.pallas import tpu_sc as plsc

_BB = 4


def _tok_body(x_ref, o_ref, tok_ref, *, C, GG):
    xb = x_ref[...]
    mx = jnp.max(xb, axis=1)
    rev = (C - jax.lax.broadcasted_iota(jnp.int32, xb.shape, 1)).astype(
        jnp.float32
    )
    hit = jnp.where(xb == mx[:, None], rev, 0.0)
    tok = (C - jnp.max(hit, axis=1)).astype(jnp.int32)
    for b in range(_BB):
        for r in range(tok.shape[1]):
            tok_ref[0, pl.ds(b * GG + r * 128, 128)] = tok[b, r, :]
    o_ref[0, 0] = tok_ref[0, :]


def _tokens(x):
    B, C, G, G2 = x.shape
    GG = G * G2
    xv = x.reshape(B, C, 8, GG // 8)
    out = pl.pallas_call(
        functools.partial(_tok_body, C=C, GG=GG),
        grid=(B // _BB,),
        in_specs=[
            pl.BlockSpec((_BB, C, 8, GG // 8), lambda b: (b, 0, 0, 0)),
        ],
        out_specs=pl.BlockSpec((1, 1, _BB * GG), lambda b: (b, 0, 0)),
        out_shape=jax.ShapeDtypeStruct((B // _BB, 1, _BB * GG), jnp.int32),
        scratch_shapes=[pltpu.VMEM((1, _BB * GG), jnp.int32)],
        compiler_params=pltpu.CompilerParams(
            dimension_semantics=("parallel",)
        ),
    )(xv)
    return out.reshape(B * GG)


def _sc_gather(dictionary, idx):
    V, D = dictionary.shape
    (N,) = idx.shape
    info = plsc.get_sparse_core_info()
    NW = info.num_cores * info.num_subcores
    b_per_w = N // NW
    CH = 8
    chunk = b_per_w // CH
    mesh = plsc.VectorSubcoreMesh(core_axis_name="c", subcore_axis_name="s")

    @functools.partial(
        pl.kernel,
        mesh=mesh,
        out_type=jax.ShapeDtypeStruct((N, D), jnp.float32),
        scratch_types=[
            pltpu.VMEM((chunk,), jnp.int32),
            pltpu.VMEM((chunk, D), jnp.float32),
            pltpu.SemaphoreType.DMA,
        ],
    )
    def k(table_hbm, idx_hbm, out_hbm, idx_v, rows_v, sem):
        wid = lax.axis_index("s") * info.num_cores + lax.axis_index("c")
        base = wid * b_per_w
        for j in range(CH):
            off = base + j * chunk
            pltpu.sync_copy(idx_hbm.at[pl.ds(off, chunk)], idx_v)
            pltpu.async_copy(table_hbm.at[idx_v], rows_v, sem).wait()
            pltpu.sync_copy(rows_v, out_hbm.at[pl.ds(off, chunk)])

    return k(dictionary, idx)


def kernel(x, dictionary):
    B, C, G, G2 = x.shape
    E = dictionary.shape[1]
    GG = G * G2
    tok = _tokens(x)
    dpad = jnp.pad(dictionary, ((0, 0), (0, 128 - E)))
    rows = _sc_gather(dpad, tok)
    rows = rows[:, :E]
    return rows.reshape(B, GG, E).transpose(0, 2, 1).reshape(B, E, G, G2)


# speedup vs baseline: 1.0665x; 1.0665x over previous
import functools

import jax
import jax.numpy as jnp
from jax.experimental import pallas as pl
from jax.experimental.pallas import tpu as pltpu

_BB = 4


def _body(x_ref, dt_ref, o_ref, mx_ref, tok0_ref, tok_ref, *, C, GG):
    k = pl.program_id(1)
    xb = x_ref[...]
    Ch = xb.shape[1]
    mxk = jnp.max(xb, axis=1)
    base = C - k * Ch
    rev = (base - jax.lax.broadcasted_iota(jnp.int32, xb.shape, 1)).astype(
        jnp.float32
    )
    hit = jnp.where(xb == mxk[:, None], rev, 0.0)
    tokk = (C - jnp.max(hit, axis=1)).astype(jnp.int32)

    @pl.when(k == 0)
    def _():
        mx_ref[...] = mxk
        tok0_ref[...] = tokk

    @pl.when(k == 1)
    def _():
        mx0 = mx_ref[...]
        tok = jnp.where(mx0 >= mxk, tok0_ref[...], tokk)
        for b in range(_BB):
            for r in range(tok.shape[1]):
                tok_ref[0, pl.ds(b * GG + r * 128, 128)] = tok[b, r, :]
        tok_row = tok_ref[0, :][None, :]
        iota2 = jax.lax.broadcasted_iota(jnp.int32, (C, _BB * GG), 0)
        onehot = jnp.where(iota2 == tok_row, 1.0, 0.0).astype(
            jnp.bfloat16
        )
        mm = jax.lax.dot(
            dt_ref[...], onehot, preferred_element_type=jnp.float32
        )
        for b in range(_BB):
            o_ref[b] = mm[:, b * GG : (b + 1) * GG]


def kernel(x, dictionary):
    B, C, G, G2 = x.shape
    E = dictionary.shape[1]
    GG = G * G2
    xv = x.reshape(B, C, 8, GG // 8)
    dict_t = dictionary.T.astype(jnp.bfloat16)
    out = pl.pallas_call(
        functools.partial(_body, C=C, GG=GG),
        grid=(B // _BB, 2),
        in_specs=[
            pl.BlockSpec(
                (_BB, C // 2, 8, GG // 8), lambda g, k: (g, k, 0, 0)
            ),
            pl.BlockSpec((E, C), lambda g, k: (0, 0)),
        ],
        out_specs=pl.BlockSpec((_BB, E, GG), lambda g, k: (g, 0, 0)),
        out_shape=jax.ShapeDtypeStruct((B, E, GG), jnp.float32),
        scratch_shapes=[
            pltpu.VMEM((_BB, 8, GG // 8), jnp.float32),
            pltpu.VMEM((_BB, 8, GG // 8), jnp.int32),
            pltpu.VMEM((1, _BB * GG), jnp.int32),
        ],
        compiler_params=pltpu.CompilerParams(
            dimension_semantics=("parallel", "arbitrary")
        ),
    )(xv, dict_t)
    return out.reshape(B, E, G, G2)


# speedup vs baseline: 1.1065x; 1.0375x over previous
import functools

import jax
import jax.numpy as jnp
from jax.experimental import pallas as pl
from jax.experimental.pallas import tpu as pltpu

_BB = 4


def _body(x_ref, dt_ref, o_ref, tok_ref, *, C, GG):
    xb = x_ref[...]
    mx = jnp.max(xb, axis=1)
    rev = (C - jax.lax.broadcasted_iota(jnp.int32, xb.shape, 1)).astype(
        jnp.float32
    )
    hit = jnp.where(xb == mx[:, None], rev, 0.0)
    tok = (C - jnp.max(hit, axis=1)).astype(jnp.int32)
    for b in range(_BB):
        for r in range(tok.shape[1]):
            tok_ref[0, pl.ds(b * GG + r * 128, 128)] = tok[b, r, :]
    tok_row = tok_ref[0, :][None, :]
    iota2 = jax.lax.broadcasted_iota(jnp.int32, (C, _BB * GG), 0)
    onehot = jnp.where(iota2 == tok_row, 1.0, 0.0).astype(
        jnp.bfloat16
    )
    mm = jax.lax.dot(
        dt_ref[...], onehot, preferred_element_type=jnp.float32
    )
    for b in range(_BB):
        o_ref[b] = mm[:, b * GG : (b + 1) * GG]


def kernel(x, dictionary):
    B, C, G, G2 = x.shape
    E = dictionary.shape[1]
    GG = G * G2
    xv = x.reshape(B, C, 8, GG // 8)
    dict_t = dictionary.T.astype(jnp.bfloat16)
    out = pl.pallas_call(
        functools.partial(_body, C=C, GG=GG),
        grid=(B // _BB,),
        in_specs=[
            pl.BlockSpec((_BB, C, 8, GG // 8), lambda b: (b, 0, 0, 0)),
            pl.BlockSpec((E, C), lambda b: (0, 0)),
        ],
        out_specs=pl.BlockSpec((_BB, E, GG), lambda b: (b, 0, 0)),
        out_shape=jax.ShapeDtypeStruct((B, E, GG), jnp.float32),
        scratch_shapes=[pltpu.VMEM((1, _BB * GG), jnp.int32)],
        compiler_params=pltpu.CompilerParams(
            dimension_semantics=("parallel",)
        ),
    )(xv, dict_t)
    return out.reshape(B, E, G, G2)
